# baseline (speedup 1.0000x reference)
import functools

import jax
import jax.numpy as jnp
from jax import lax
from jax.experimental import pallas as pl
from jax.experimental.pallas import tpu as pltpu
from jax.experimental.pallas import tpu_sc as plsc

B = 16384
D = 64
NH = 128
NW = 32
RPW = B // NW
NCHUNK = 4
CH = RPW // NCHUNK
L = 16
GPC = CH // L


def _sc_kernel_body(tidx_hbm, midx_hbm, tt_hbm, mt_hbm, w_hbm, out_hbm,
                    idx_t, idx_m, idx2_t, idx2_m, et, em, wv, ov, *sems):
    wid = lax.axis_index("s") * 2 + lax.axis_index("c")
    pltpu.sync_copy(tidx_hbm.at[wid], idx_t)
    pltpu.sync_copy(midx_hbm.at[wid], idx_m)
    pltpu.sync_copy(w_hbm, wv)

    for c in range(NCHUNK):
        for s in range(CH // L):
            sl = pl.ds(s * L, L)
            idx2_t[c, sl] = lax.shift_right_logical(idx_t[c, sl], 1)
            idx2_m[c, sl] = lax.shift_right_logical(idx_m[c, sl], 1)

    def gather(c, slot, sem_pair):
        cp_t = pltpu.async_copy(tt_hbm.at[idx2_t.at[c]], et.at[slot],
                                sem_pair[0])
        cp_m = pltpu.async_copy(mt_hbm.at[idx2_m.at[c]], em.at[slot],
                                sem_pair[1])
        return cp_t, cp_m

    copies = [None, None]
    copies[0] = gather(0, 0, sems[0:2])
    copies[1] = gather(1, 1, sems[2:4])

    lane = lax.iota(jnp.int32, 16)
    lane3 = lane * 3

    for c in range(NCHUNK):
        slot = c % 2
        copies[slot][0].wait()
        copies[slot][1].wait()

        def group_body(g, carry):
            offt = lax.shift_left(lax.bitwise_and(idx_t[c, pl.ds(g * L, L)],
                                                  1), 6)
            offm = lax.shift_left(lax.bitwise_and(idx_m[c, pl.ds(g * L, L)],
                                                  1), 6)
            row = lane + g * L
            acc0 = jnp.zeros((L,), jnp.float32)
            acc1 = jnp.zeros((L,), jnp.float32)
            acc2 = jnp.zeros((L,), jnp.float32)
            for d in range(D):
                colt = offt + d
                colm = offm + d
                etv = plsc.load_gather(et.at[slot], [row, colt])
                emv = plsc.load_gather(em.at[slot], [row, colm])
                pv = etv * emv
                wrow = wv[d]
                acc0 = acc0 + etv * wrow[0] + emv * wrow[3] + pv * wrow[6]
                acc1 = acc1 + etv * wrow[1] + emv * wrow[4] + pv * wrow[7]
                acc2 = acc2 + etv * wrow[2] + emv * wrow[5] + pv * wrow[8]
            ob = (c * CH + g * L) * 3
            plsc.store_scatter(ov, [lane3 + ob], acc0)
            plsc.store_scatter(ov, [lane3 + (ob + 1)], acc1)
            plsc.store_scatter(ov, [lane3 + (ob + 2)], acc2)
            return carry

        lax.fori_loop(0, GPC, group_body, 0)
        if c + 2 < NCHUNK:
            copies[slot] = gather(c + 2, slot, sems[4 + 2 * slot:6 + 2 * slot])

    pltpu.sync_copy(ov, out_hbm.at[pl.ds(wid * (3 * RPW), 3 * RPW)])


@jax.jit
def _sc_part(tidx, midx, tt2, mt2, w_pack):
    mesh = plsc.VectorSubcoreMesh(core_axis_name="c", subcore_axis_name="s")
    scratch = [
        pltpu.VMEM((NCHUNK, CH), jnp.int32),
        pltpu.VMEM((NCHUNK, CH), jnp.int32),
        pltpu.VMEM((NCHUNK, CH), jnp.int32),
        pltpu.VMEM((NCHUNK, CH), jnp.int32),
        pltpu.VMEM((2, CH, 2 * D), jnp.float32),
        pltpu.VMEM((2, CH, 2 * D), jnp.float32),
        pltpu.VMEM((D, L), jnp.float32),
        pltpu.VMEM((3 * RPW,), jnp.float32),
    ] + [pltpu.SemaphoreType.DMA] * 8
    return pl.kernel(
        _sc_kernel_body,
        mesh=mesh,
        out_type=jax.ShapeDtypeStruct((B * 3,), jnp.float32),
        scratch_types=scratch,
        compiler_params=pltpu.CompilerParams(
            needs_layout_passes=False, use_tc_tiling_on_sc=False),
    )(tidx, midx, tt2, mt2, w_pack)


def _tc_body(h_ref, w_ref, b_ref, o_ref):
    o_ref[...] = jnp.dot(h_ref[...], w_ref[...],
                         preferred_element_type=jnp.float32) + b_ref[...]


@jax.jit
def _tc_part(h, w_h, b2):
    blk = 2048
    return pl.pallas_call(
        _tc_body,
        grid=(B // blk,),
        in_specs=[
            pl.BlockSpec((blk, NH), lambda i: (i, 0)),
            pl.BlockSpec((NH, 3), lambda i: (0, 0)),
            pl.BlockSpec((1, 3), lambda i: (0, 0)),
        ],
        out_specs=pl.BlockSpec((blk, 3), lambda i: (i, 0)),
        out_shape=jax.ShapeDtypeStruct((B, 3), jnp.float32),
    )(h, w_h, b2)


def kernel(h, teacher_idx, materia_idx, teacher_table, materia_table, W, b):
    tidx = teacher_idx.astype(jnp.int32).reshape(NW, NCHUNK, CH)
    midx = materia_idx.astype(jnp.int32).reshape(NW, NCHUNK, CH)
    tt2 = teacher_table.reshape(-1, 2 * D)
    mt2 = materia_table.reshape(-1, 2 * D)
    w_pack = jnp.concatenate(
        [W[NH:NH + D], W[NH + D:NH + 2 * D], W[NH + 2 * D:],
         jnp.zeros((D, 7), jnp.float32)], axis=1)
    sc = _sc_part(tidx, midx, tt2, mt2, w_pack)
    tc = _tc_part(h, W[:NH], b.reshape(1, 3))
    return tc + sc.reshape(B, 3)
